# Initial kernel scaffold; baseline (speedup 1.0000x reference)
import functools

import jax
import jax.numpy as jnp
from jax import lax
from jax.experimental import pallas as pl
from jax.experimental.pallas import tpu as pltpu
from jax.experimental.pallas import tpu_sc as plsc

F = 128
R = 16
C_ALL = 9 * F
G2_W = 400
NODE_CHUNK = 1024
E_BLK = 32


def _silu(t):
    return t / (1.0 + jnp.exp(-t))


def _res_mlp(t, t0, w1, b1, w2, b2, wo, bo):
    h = jax.lax.dot_general(t0, w1, (((1,), (1,)), ((), ())),
                            preferred_element_type=jnp.float32) + b1
    h = _silu(h)
    h = jax.lax.dot_general(h, w2, (((1,), (1,)), ((), ())),
                            preferred_element_type=jnp.float32) + b2
    u = _silu(t + h)
    return jax.lax.dot_general(u, wo, (((1,), (1,)), ((), ())),
                               preferred_element_type=jnp.float32) + bo


def _pre_a_body(x_ref, *refs):
    (xw1, xb1, xw2, xb2, xwo, xbo,
     sw1, sb1, sw2, sb2, swo, sbo,
     pw1, pb1, pw2, pb2, pwo, pbo,
     dw1, db1, dw2, db2, dwo, dbo,
     ycat_ref, xx_ref) = refs
    x = x_ref[...]
    t0 = _silu(x)
    xx_ref[...] = _res_mlp(x, t0, xw1[...], xb1[0], xw2[...], xb2[0],
                           xwo[...], xbo[0])
    ycat_ref[:, 0:F] = _res_mlp(x, t0, sw1[...], sb1[0], sw2[...], sb2[0],
                                swo[...], sbo[0])
    ycat_ref[:, F:2 * F] = _res_mlp(x, t0, pw1[...], pb1[0], pw2[...], pb2[0],
                                    pwo[...], pbo[0])
    ycat_ref[:, 2 * F:3 * F] = _res_mlp(x, t0, dw1[...], db1[0], dw2[...],
                                        db2[0], dwo[...], dbo[0])


def _run_pre_a(x, mlp_x, mlp_s, mlp_p, mlp_d):
    n = x.shape[0]
    bn = 400
    grid = n // bn
    wspec = pl.BlockSpec((F, F), lambda i: (0, 0))
    bspec = pl.BlockSpec((1, F), lambda i: (0, 0))
    wrefs = []
    specs = [pl.BlockSpec((bn, F), lambda i: (i, 0))]
    for m in (mlp_x, mlp_s, mlp_p, mlp_d):
        for key in ("w1", "b1", "w2", "b2", "wo", "bo"):
            v = m[key]
            if v.ndim == 1:
                wrefs.append(v.reshape(1, F))
                specs.append(bspec)
            else:
                wrefs.append(v)
                specs.append(wspec)
    return pl.pallas_call(
        _pre_a_body,
        grid=(grid,),
        in_specs=specs,
        out_specs=[pl.BlockSpec((bn, 3 * F), lambda i: (i, 0)),
                   pl.BlockSpec((bn, F), lambda i: (i, 0))],
        out_shape=[jax.ShapeDtypeStruct((n, 3 * F), jnp.float32),
                   jax.ShapeDtypeStruct((n, F), jnp.float32)],
    )(x, *wrefs)


def _pre_b_body(rbf_ref, tail_ref, wcat_ref, g2_ref):
    rbf = rbf_ref[...]
    g = jax.lax.dot_general(rbf, wcat_ref[...], (((1,), (1,)), ((), ())),
                            preferred_element_type=jnp.float32)
    g2_ref[:, 0:3 * F] = g
    g2_ref[:, 3 * F:3 * F + 16] = tail_ref[...]


def _run_pre_b(rbf, tail, wcat):
    p = rbf.shape[0]
    bp = 2000
    grid = p // bp
    return pl.pallas_call(
        _pre_b_body,
        grid=(grid,),
        in_specs=[pl.BlockSpec((bp, R), lambda i: (i, 0)),
                  pl.BlockSpec((bp, 16), lambda i: (i, 0)),
                  pl.BlockSpec((3 * F, R), lambda i: (0, 0))],
        out_specs=pl.BlockSpec((bp, G2_W), lambda i: (i, 0)),
        out_shape=jax.ShapeDtypeStruct((p, G2_W), jnp.float32),
    )(rbf, tail, wcat)


def _bounds_body(idx_ref, out_ref):
    x = idx_ref[...]
    vals = []
    for c in range(16):
        thr = c * NODE_CHUNK
        vals.append(jnp.sum((x < thr).astype(jnp.int32)))
    out_ref[0, :] = jnp.stack(vals)


def _run_bounds(idx_i32_2d):
    return pl.pallas_call(
        _bounds_body,
        out_shape=jax.ShapeDtypeStruct((1, 16), jnp.int32),
    )(idx_i32_2d)


def _post_body(xx_ref, spd_ref, pp_ref, pd_ref, *orefs):
    (w1, b1, w2, b2, wo, bo, out_ref) = orefs
    s = xx_ref[...] + spd_ref[:, 0:F]
    acc = s
    for k in range(3):
        pk = spd_ref[:, F + k * F:F + (k + 1) * F]
        prj = jax.lax.dot_general(pk, pp_ref[...], (((1,), (1,)), ((), ())),
                                  preferred_element_type=jnp.float32)
        acc = acc + prj[:, 0:F] * prj[:, F:2 * F]
    for k in range(5):
        dk = spd_ref[:, 4 * F + k * F:4 * F + (k + 1) * F]
        prj = jax.lax.dot_general(dk, pd_ref[...], (((1,), (1,)), ((), ())),
                                  preferred_element_type=jnp.float32)
        acc = acc + prj[:, 0:F] * prj[:, F:2 * F]
    t0 = _silu(acc)
    out_ref[...] = _res_mlp(acc, t0, w1[...], b1[0], w2[...], b2[0],
                            wo[...], bo[0])


def _run_post(xx, spd, pp, pd, mlp_o):
    n = xx.shape[0]
    bn = 400
    grid = n // bn
    wspec = pl.BlockSpec((F, F), lambda i: (0, 0))
    bspec = pl.BlockSpec((1, F), lambda i: (0, 0))
    wrefs = []
    specs = [pl.BlockSpec((bn, F), lambda i: (i, 0)),
             pl.BlockSpec((bn, C_ALL), lambda i: (i, 0)),
             pl.BlockSpec((2 * F, F), lambda i: (0, 0)),
             pl.BlockSpec((2 * F, F), lambda i: (0, 0))]
    for key in ("w1", "b1", "w2", "b2", "wo", "bo"):
        v = mlp_o[key]
        if v.ndim == 1:
            wrefs.append(v.reshape(1, F))
            specs.append(bspec)
        else:
            wrefs.append(v)
            specs.append(wspec)
    return pl.pallas_call(
        _post_body,
        grid=(grid,),
        in_specs=specs,
        out_specs=pl.BlockSpec((bn, F), lambda i: (i, 0)),
        out_shape=jax.ShapeDtypeStruct((n, F), jnp.float32),
    )(xx, spd, pp, pd, *wrefs)


def _sc_body(ycat, g2f, bounds, out,
             g2_buf, y_buf, c_buf, ij_buf, lidx_buf, bsmem, acc, sem):
    cid = lax.axis_index("c")
    sid = lax.axis_index("s")
    nchunk = out.shape[0] // NODE_CHUNK
    lane = lax.iota(jnp.int32, 16)

    pltpu.sync_copy(bounds, bsmem)

    def zero_c():
        zv = jnp.zeros((16,), jnp.float32)

        def zr(r, _):
            def zc(j, _):
                c_buf[r, pl.ds(j * 16, 16)] = zv
                return 0
            return lax.fori_loop(0, C_ALL // 16, zc, 0)
        lax.fori_loop(0, E_BLK, zr, 0)

    for half in range(nchunk // 2):
        chunk = half * 2 + cid
        node_base = chunk * NODE_CHUNK
        e_lo_c = bsmem[chunk]
        e_hi_c = bsmem[chunk + 1]

        zero_c()
        for h in range(NODE_CHUNK // 16 // E_BLK):
            pltpu.sync_copy(
                c_buf, acc.at[pl.ds(sid * (NODE_CHUNK // 16) + h * E_BLK,
                                    E_BLK)])
        plsc.subcore_barrier()

        nume = e_hi_c - e_lo_c
        e_lo = e_lo_c + (sid * nume) // 16
        e_hi = e_lo_c + ((sid + 1) * nume) // 16
        b_lo = e_lo // E_BLK
        nblk = (e_hi + E_BLK - 1) // E_BLK - b_lo

        def block_body(b, _):
            e0 = (b_lo + b) * E_BLK
            pltpu.sync_copy(g2f.at[pl.ds(e0 * G2_W, E_BLK * G2_W)],
                            g2_buf)
            for grp in range(E_BLK // 16):
                rb = (grp * 16 + lane) * G2_W
                ijf = plsc.load_gather(g2_buf, [rb + 393])
                ij_buf[pl.ds(grp * 16, 16)] = plsc.bitcast(ijf, jnp.int32)
            pltpu.async_copy(ycat.at[ij_buf], y_buf, sem).wait()

            for grp in range(E_BLK // 16):
                rowv = grp * 16 + lane
                rb_g2 = rowv * G2_W
                eabs = e0 + rowv
                iif = plsc.load_gather(g2_buf, [rb_g2 + 392])
                iiv = plsc.bitcast(iif, jnp.int32)
                valid = (eabs >= e_lo) & (eabs < e_hi)
                li = iiv - node_base
                li = jnp.minimum(jnp.maximum(li, 0), NODE_CHUNK - 1)
                lidx_buf[pl.ds(grp * 16, 16)] = jnp.where(valid, li, 0)
                maskf = jnp.where(valid, 1.0, 0.0).astype(jnp.float32)
                wv = [plsc.load_gather(g2_buf, [rb_g2 + (3 * F + k)])
                      for k in range(8)]

                def f_body(f, _):
                    fv = jnp.full((16,), 0, jnp.int32) + f
                    ys = plsc.load_gather(y_buf, [rowv, fv])
                    yp = plsc.load_gather(y_buf, [rowv, fv + F])
                    yd = plsc.load_gather(y_buf, [rowv, fv + 2 * F])
                    gs = plsc.load_gather(g2_buf, [rb_g2 + fv])
                    gp = plsc.load_gather(g2_buf, [rb_g2 + (fv + F)])
                    gd = plsc.load_gather(g2_buf, [rb_g2 + (fv + 2 * F)])
                    ms = ys * gs * maskf
                    mp = yp * gp * maskf
                    md = yd * gd * maskf
                    plsc.store_scatter(c_buf, [rowv, fv], ms)
                    for k in range(3):
                        plsc.store_scatter(c_buf, [rowv, fv + F + k * F],
                                           mp * wv[k])
                    for k in range(5):
                        plsc.store_scatter(c_buf, [rowv, fv + 4 * F + k * F],
                                           md * wv[3 + k])
                    return 0

                lax.fori_loop(0, F, f_body, 0)

            pltpu.sync_copy(c_buf, acc.at[lidx_buf], add=True)
            return 0

        lax.fori_loop(0, nblk, block_body, 0)
        plsc.subcore_barrier()

        for h in range(NODE_CHUNK // 16 // E_BLK):
            r0 = sid * (NODE_CHUNK // 16) + h * E_BLK
            pltpu.sync_copy(acc.at[pl.ds(r0, E_BLK)], c_buf)
            pltpu.sync_copy(c_buf, out.at[pl.ds(node_base + r0, E_BLK)])
        plsc.subcore_barrier()


def _run_sc(ycat, g2, bounds, n_pad):
    mesh = plsc.VectorSubcoreMesh(core_axis_name="c", subcore_axis_name="s")
    fn = pl.kernel(
        _sc_body,
        mesh=mesh,
        out_type=jax.ShapeDtypeStruct((n_pad, C_ALL), jnp.float32),
        scratch_types=[
            pltpu.VMEM((E_BLK, G2_W), jnp.float32),
            pltpu.VMEM((E_BLK, 3 * F), jnp.float32),
            pltpu.VMEM((E_BLK, C_ALL), jnp.float32),
            pltpu.VMEM((E_BLK,), jnp.int32),
            pltpu.VMEM((E_BLK,), jnp.int32),
            pltpu.SMEM((16,), jnp.int32),
            pltpu.VMEM_SHARED((NODE_CHUNK, C_ALL), jnp.float32),
            pltpu.SemaphoreType.DMA,
        ],
    )
    return fn(ycat, g2.reshape(-1), bounds)


def kernel(x, rbf, pij, dij, idx_i, idx_j, Ws, Wp, Wd, Pp, Pd,
           mlp_x, mlp_s, mlp_p, mlp_d, mlp_o):
    n = x.shape[0]
    p = rbf.shape[0]
    n_pad = ((n + NODE_CHUNK - 1) // NODE_CHUNK) * NODE_CHUNK

    ii32 = idx_i.astype(jnp.int32)
    ij32 = idx_j.astype(jnp.int32)
    tail = jnp.concatenate([
        pij, dij,
        jax.lax.bitcast_convert_type(ii32, jnp.float32)[:, None],
        jax.lax.bitcast_convert_type(ij32, jnp.float32)[:, None],
        jnp.zeros((p, 6), jnp.float32),
    ], axis=1)
    wcat = jnp.concatenate([Ws, Wp, Wd], axis=0)

    ycat, xx = _run_pre_a(x, mlp_x, mlp_s, mlp_p, mlp_d)
    g2 = _run_pre_b(rbf, tail, wcat)
    bounds = _run_bounds(ii32.reshape(p // F, F)).reshape(16)
    spd = _run_sc(ycat, g2, bounds, n_pad)
    return _run_post(xx, spd[:n], Pp, Pd, mlp_o)


# speedup vs baseline: 16.7460x; 16.7460x over previous
import jax
import jax.numpy as jnp
from jax import lax
from jax.experimental import pallas as pl
from jax.experimental.pallas import tpu as pltpu
from jax.experimental.pallas import tpu_sc as plsc

F = 128
R = 16
C_ALL = 9 * F
G2_W = 400
NODE_CHUNK = 64
NBOUND = 256
E_BLK = 32


def _silu(t):
    return t / (1.0 + jnp.exp(-t))


def _res_mlp(t, t0, w1, b1, w2, b2, wo, bo):
    h = jax.lax.dot_general(t0, w1, (((1,), (1,)), ((), ())),
                            preferred_element_type=jnp.float32) + b1
    h = _silu(h)
    h = jax.lax.dot_general(h, w2, (((1,), (1,)), ((), ())),
                            preferred_element_type=jnp.float32) + b2
    u = _silu(t + h)
    return jax.lax.dot_general(u, wo, (((1,), (1,)), ((), ())),
                               preferred_element_type=jnp.float32) + bo


def _pre_a_body(x_ref, *refs):
    (xw1, xb1, xw2, xb2, xwo, xbo,
     sw1, sb1, sw2, sb2, swo, sbo,
     pw1, pb1, pw2, pb2, pwo, pbo,
     dw1, db1, dw2, db2, dwo, dbo,
     ycat_ref, xx_ref) = refs
    x = x_ref[...]
    t0 = _silu(x)
    xx_ref[...] = _res_mlp(x, t0, xw1[...], xb1[0], xw2[...], xb2[0],
                           xwo[...], xbo[0])
    ycat_ref[:, 0:F] = _res_mlp(x, t0, sw1[...], sb1[0], sw2[...], sb2[0],
                                swo[...], sbo[0])
    ycat_ref[:, F:2 * F] = _res_mlp(x, t0, pw1[...], pb1[0], pw2[...], pb2[0],
                                    pwo[...], pbo[0])
    ycat_ref[:, 2 * F:3 * F] = _res_mlp(x, t0, dw1[...], db1[0], dw2[...],
                                        db2[0], dwo[...], dbo[0])


def _run_pre_a(x, mlp_x, mlp_s, mlp_p, mlp_d):
    n = x.shape[0]
    bn = 400
    grid = n // bn
    wspec = pl.BlockSpec((F, F), lambda i: (0, 0))
    bspec = pl.BlockSpec((1, F), lambda i: (0, 0))
    wrefs = []
    specs = [pl.BlockSpec((bn, F), lambda i: (i, 0))]
    for m in (mlp_x, mlp_s, mlp_p, mlp_d):
        for key in ("w1", "b1", "w2", "b2", "wo", "bo"):
            v = m[key]
            if v.ndim == 1:
                wrefs.append(v.reshape(1, F))
                specs.append(bspec)
            else:
                wrefs.append(v)
                specs.append(wspec)
    return pl.pallas_call(
        _pre_a_body,
        grid=(grid,),
        in_specs=specs,
        out_specs=[pl.BlockSpec((bn, 3 * F), lambda i: (i, 0)),
                   pl.BlockSpec((bn, F), lambda i: (i, 0))],
        out_shape=[jax.ShapeDtypeStruct((n, 3 * F), jnp.float32),
                   jax.ShapeDtypeStruct((n, F), jnp.float32)],
    )(x, *wrefs)


def _pre_b_body(rbf_ref, tail_ref, wcat_ref, g2_ref):
    rbf = rbf_ref[...]
    g = jax.lax.dot_general(rbf, wcat_ref[...], (((1,), (1,)), ((), ())),
                            preferred_element_type=jnp.float32)
    g2_ref[:, 0:3 * F] = g
    g2_ref[:, 3 * F:3 * F + 16] = tail_ref[...]


def _run_pre_b(rbf, tail, wcat):
    p = rbf.shape[0]
    bp = 2000
    grid = p // bp
    return pl.pallas_call(
        _pre_b_body,
        grid=(grid,),
        in_specs=[pl.BlockSpec((bp, R), lambda i: (i, 0)),
                  pl.BlockSpec((bp, 16), lambda i: (i, 0)),
                  pl.BlockSpec((3 * F, R), lambda i: (0, 0))],
        out_specs=pl.BlockSpec((bp, G2_W), lambda i: (i, 0)),
        out_shape=jax.ShapeDtypeStruct((p, G2_W), jnp.float32),
    )(rbf, tail, wcat)


def _bounds_body(idx_ref, out_ref):
    nrow = idx_ref.shape[0]
    thr = lax.broadcasted_iota(jnp.int32, (1, NBOUND), 1) * NODE_CHUNK

    def body(r, acc):
        row = idx_ref[r, :]
        cmp = (row[:, None] < thr[0][None, :]).astype(jnp.int32)
        return acc + jnp.sum(cmp, axis=0, keepdims=True)

    out_ref[...] = lax.fori_loop(0, nrow, body,
                                 jnp.zeros((1, NBOUND), jnp.int32))


def _run_bounds(idx_i32_2d):
    return pl.pallas_call(
        _bounds_body,
        out_shape=jax.ShapeDtypeStruct((1, NBOUND), jnp.int32),
    )(idx_i32_2d)


def _post_body(xx_ref, spd_ref, pp_ref, pd_ref, *orefs):
    (w1, b1, w2, b2, wo, bo, out_ref) = orefs
    s = xx_ref[...] + spd_ref[:, 0:F]
    acc = s
    for k in range(3):
        pk = spd_ref[:, F + k * F:F + (k + 1) * F]
        prj = jax.lax.dot_general(pk, pp_ref[...], (((1,), (1,)), ((), ())),
                                  preferred_element_type=jnp.float32)
        acc = acc + prj[:, 0:F] * prj[:, F:2 * F]
    for k in range(5):
        dk = spd_ref[:, 4 * F + k * F:4 * F + (k + 1) * F]
        prj = jax.lax.dot_general(dk, pd_ref[...], (((1,), (1,)), ((), ())),
                                  preferred_element_type=jnp.float32)
        acc = acc + prj[:, 0:F] * prj[:, F:2 * F]
    t0 = _silu(acc)
    out_ref[...] = _res_mlp(acc, t0, w1[...], b1[0], w2[...], b2[0],
                            wo[...], bo[0])


def _run_post(xx, spd, pp, pd, mlp_o):
    n = xx.shape[0]
    bn = 400
    grid = n // bn
    wspec = pl.BlockSpec((F, F), lambda i: (0, 0))
    bspec = pl.BlockSpec((1, F), lambda i: (0, 0))
    wrefs = []
    specs = [pl.BlockSpec((bn, F), lambda i: (i, 0)),
             pl.BlockSpec((bn, C_ALL), lambda i: (i, 0)),
             pl.BlockSpec((2 * F, F), lambda i: (0, 0)),
             pl.BlockSpec((2 * F, F), lambda i: (0, 0))]
    for key in ("w1", "b1", "w2", "b2", "wo", "bo"):
        v = mlp_o[key]
        if v.ndim == 1:
            wrefs.append(v.reshape(1, F))
            specs.append(bspec)
        else:
            wrefs.append(v)
            specs.append(wspec)
    return pl.pallas_call(
        _post_body,
        grid=(grid,),
        in_specs=specs,
        out_specs=pl.BlockSpec((bn, F), lambda i: (i, 0)),
        out_shape=jax.ShapeDtypeStruct((n, F), jnp.float32),
    )(xx, spd, pp, pd, *wrefs)


ACCW = NODE_CHUNK * C_ALL


def _splat(v, e):
    idx = jnp.full((16,), 0, jnp.int32) + e
    return jnp.take_along_axis(v, idx, axis=0)


def _sc_body(ycat, g2f, bounds, outf,
             g2_buf, y_buf, ij_buf, bnd_buf, acc, sem):
    cid = lax.axis_index("c")
    sid = lax.axis_index("s")
    wid = sid * 2 + cid
    lane = lax.iota(jnp.int32, 16)
    nchunk_w = outf.shape[0] // ACCW // 32

    pltpu.sync_copy(bounds, bnd_buf)

    def _bnd(i):
        base = (i // 16) * 16
        v = bnd_buf[pl.ds(base, 16)]
        return jnp.sum(jnp.where(lane == (i - base), v, 0))

    def zero_acc():
        zv = jnp.zeros((16,), jnp.float32)

        def z(i, _):
            acc[pl.ds(i * 16, 16)] = zv
            return 0
        lax.fori_loop(0, ACCW // 16, z, 0)

    for q in range(nchunk_w):
        chunk = wid * nchunk_w + q
        node_base = chunk * NODE_CHUNK
        e_lo = _bnd(chunk)
        e_hi = _bnd(chunk + 1)
        zero_acc()
        b_lo = e_lo // E_BLK
        nblk = (e_hi + E_BLK - 1) // E_BLK - b_lo

        def block_body(b, _):
            e0 = (b_lo + b) * E_BLK
            pltpu.sync_copy(g2f.at[pl.ds(e0 * G2_W, E_BLK * G2_W)], g2_buf)
            for grp in range(E_BLK // 16):
                rb = (grp * 16 + lane) * G2_W
                ijf = plsc.load_gather(g2_buf, [rb + 393])
                ij_buf[pl.ds(grp * 16, 16)] = plsc.bitcast(ijf, jnp.int32)
            pltpu.async_copy(ycat.at[ij_buf], y_buf, sem).wait()

            for grp in range(E_BLK // 16):
                g0 = grp * 16
                rb_g2 = (g0 + lane) * G2_W
                iif = plsc.load_gather(g2_buf, [rb_g2 + 392])
                iiv = plsc.bitcast(iif, jnp.int32)
                li = iiv - node_base
                li = jnp.minimum(jnp.maximum(li, 0), NODE_CHUNK - 1)
                rowbase = li * C_ALL
                wv = [plsc.load_gather(g2_buf, [rb_g2 + (3 * F + k)])
                      for k in range(8)]
                sb = jnp.clip(e_lo - (e0 + g0), 0, 16)
                eb = jnp.clip(e_hi - (e0 + g0), 0, 16)

                def e_body(e, _):
                    ge = g0 + e
                    be = ge * G2_W
                    rs = _splat(rowbase, e) + lane
                    ws = [_splat(w, e) for w in wv]
                    for j in range(F // 16):
                        o = 16 * j
                        ys = y_buf[ge, pl.ds(o, 16)]
                        gs = g2_buf[pl.ds(be + o, 16)]
                        plsc.addupdate_scatter(acc, [rs + o], ys * gs)
                        yp = y_buf[ge, pl.ds(F + o, 16)]
                        gp = g2_buf[pl.ds(be + F + o, 16)]
                        mp = yp * gp
                        for k in range(3):
                            plsc.addupdate_scatter(
                                acc, [rs + (F + k * F + o)], mp * ws[k])
                        yd = y_buf[ge, pl.ds(2 * F + o, 16)]
                        gd = g2_buf[pl.ds(be + 2 * F + o, 16)]
                        md = yd * gd
                        for k in range(5):
                            plsc.addupdate_scatter(
                                acc, [rs + (4 * F + k * F + o)],
                                md * ws[3 + k])
                    return 0

                lax.fori_loop(sb, eb, e_body, 0)
            return 0

        lax.fori_loop(0, nblk, block_body, 0)

        pltpu.sync_copy(acc, outf.at[pl.ds(chunk * ACCW, ACCW)])


def _run_sc(ycat, g2, bounds, n_pad):
    mesh = plsc.VectorSubcoreMesh(core_axis_name="c", subcore_axis_name="s")
    fn = pl.kernel(
        _sc_body,
        mesh=mesh,
        compiler_params=pltpu.CompilerParams(needs_layout_passes=False),
        out_type=jax.ShapeDtypeStruct((n_pad * C_ALL,), jnp.float32),
        scratch_types=[
            pltpu.VMEM((E_BLK * G2_W,), jnp.float32),
            pltpu.VMEM((E_BLK, 3 * F), jnp.float32),
            pltpu.VMEM((E_BLK,), jnp.int32),
            pltpu.VMEM((NBOUND,), jnp.int32),
            pltpu.VMEM((ACCW,), jnp.float32),
            pltpu.SemaphoreType.DMA,
        ],
    )
    return fn(ycat, g2.reshape(-1), bounds)


def kernel(x, rbf, pij, dij, idx_i, idx_j, Ws, Wp, Wd, Pp, Pd,
           mlp_x, mlp_s, mlp_p, mlp_d, mlp_o):
    n = x.shape[0]
    p = rbf.shape[0]
    gran = NODE_CHUNK * 32
    n_pad = ((n + gran - 1) // gran) * gran

    ii32 = idx_i.astype(jnp.int32)
    ij32 = idx_j.astype(jnp.int32)
    tail = jnp.concatenate([
        pij, dij,
        jax.lax.bitcast_convert_type(ii32, jnp.float32)[:, None],
        jax.lax.bitcast_convert_type(ij32, jnp.float32)[:, None],
        jnp.zeros((p, 6), jnp.float32),
    ], axis=1)
    wcat = jnp.concatenate([Ws, Wp, Wd], axis=0)

    ycat, xx = _run_pre_a(x, mlp_x, mlp_s, mlp_p, mlp_d)
    g2 = _run_pre_b(rbf, tail, wcat)
    bounds = _run_bounds(ii32.reshape(p // F, F)).reshape(NBOUND)
    spd = _run_sc(ycat, g2, bounds, n_pad).reshape(n_pad, C_ALL)
    return _run_post(xx, spd[:n], Pp, Pd, mlp_o)
